# Initial kernel scaffold; baseline (speedup 1.0000x reference)
import jax
import jax.numpy as jnp
from jax.experimental import pallas as pl


def kernel(x, W_enc, W_dec, b_enc, b_dec, pos_emb):
    raise NotImplementedError("write your pallas kernel here")


# speedup vs baseline: 2.2140x; 2.2140x over previous
import functools
import jax
import jax.numpy as jnp
from jax.experimental import pallas as pl
from jax.experimental.pallas import tpu as pltpu

_B, _T, _DIN, _DSAE, _K = 64, 8, 256, 16384, 64


def _encode_body(x2_ref, pe2_ref, w_ref, b_ref, pre_ref):
    xs = x2_ref[:, 0:_DIN] + pe2_ref[:, 0:_DIN]
    for t in range(1, _T):
        xs = xs + x2_ref[:, t * _DIN:(t + 1) * _DIN] + pe2_ref[:, t * _DIN:(t + 1) * _DIN]
    pre_ref[...] = jnp.dot(xs, w_ref[...], preferred_element_type=jnp.float32) + b_ref[...]


def _encode(x2, pe2, W_enc, b2):
    TS = 2048
    grid = (_DSAE // TS,)
    return pl.pallas_call(
        _encode_body,
        grid=grid,
        in_specs=[
            pl.BlockSpec((_B, _T * _DIN), lambda i: (0, 0)),
            pl.BlockSpec((1, _T * _DIN), lambda i: (0, 0)),
            pl.BlockSpec((_DIN, TS), lambda i: (0, i)),
            pl.BlockSpec((1, TS), lambda i: (0, i)),
        ],
        out_specs=pl.BlockSpec((_B, TS), lambda i: (0, i)),
        out_shape=jax.ShapeDtypeStruct((_B, _DSAE), jnp.float32),
    )(x2, pe2, W_enc, b2)


def _select_body(pre_ref, z_ref):
    pre = pre_ref[...]
    bits = jax.lax.bitcast_convert_type(pre, jnp.uint32)
    key = jnp.where(bits >> 31, ~bits, bits | jnp.uint32(0x80000000))

    def step(it, lo):
        cand = lo | (jnp.uint32(1) << (jnp.uint32(31) - it.astype(jnp.uint32)))
        cnt = jnp.sum((key >= cand).astype(jnp.int32), axis=1, keepdims=True)
        return jnp.where(cnt >= _K, cand, lo)

    lo = jax.lax.fori_loop(0, 32, step, jnp.zeros((_B, 1), jnp.uint32))
    take = key >= lo
    z_ref[...] = jnp.where(take, jnp.maximum(pre, 0.0), 0.0)


def _select(pre):
    return pl.pallas_call(
        _select_body,
        out_shape=jax.ShapeDtypeStruct((_B, _DSAE), jnp.float32),
    )(pre)


def _decode_body(z_ref, w_ref, bd2_ref, x2_ref, xhat_ref, loss_ref, acc_ref):
    i = pl.program_id(0)
    n = pl.num_programs(0)

    @pl.when(i == 0)
    def _():
        acc_ref[...] = jnp.zeros_like(acc_ref)

    acc_ref[...] += jnp.dot(z_ref[...], w_ref[...], preferred_element_type=jnp.float32)

    @pl.when(i == n - 1)
    def _():
        xhat = acc_ref[...] + bd2_ref[...]
        xhat_ref[...] = xhat
        d = xhat - x2_ref[...]
        loss_ref[...] = (jnp.sum(d * d) / (_B * _T)).reshape(1, 1)


def _decode(z, W2, bd2, x2):
    CS = 2048
    grid = (_DSAE // CS,)
    return pl.pallas_call(
        _decode_body,
        grid=grid,
        in_specs=[
            pl.BlockSpec((_B, CS), lambda i: (0, i)),
            pl.BlockSpec((CS, _T * _DIN), lambda i: (i, 0)),
            pl.BlockSpec((1, _T * _DIN), lambda i: (0, 0)),
            pl.BlockSpec((_B, _T * _DIN), lambda i: (0, 0)),
        ],
        out_specs=[
            pl.BlockSpec((_B, _T * _DIN), lambda i: (0, 0)),
            pl.BlockSpec((1, 1), lambda i: (0, 0)),
        ],
        out_shape=[
            jax.ShapeDtypeStruct((_B, _T * _DIN), jnp.float32),
            jax.ShapeDtypeStruct((1, 1), jnp.float32),
        ],
        scratch_shapes=[pltpu.VMEM((_B, _T * _DIN), jnp.float32)],
    )(z, W2, bd2, x2)


def kernel(x, W_enc, W_dec, b_enc, b_dec, pos_emb):
    x2 = x.reshape(_B, _T * _DIN)
    pe2 = pos_emb.reshape(1, _T * _DIN)
    b2 = b_enc.reshape(1, _DSAE)
    W2 = W_dec.reshape(_DSAE, _T * _DIN)
    bd2 = b_dec.reshape(1, _T * _DIN)

    pre = _encode(x2, pe2, W_enc, b2)
    z = _select(pre)
    xhat2, loss = _decode(z, W2, bd2, x2)
    return (loss.reshape(()), xhat2.reshape(_B, _T, _DIN), z)
